# Initial kernel scaffold; baseline (speedup 1.0000x reference)
import jax
import jax.numpy as jnp
from jax.experimental import pallas as pl


def kernel(x, edge_index, W_self0, W_neigh0, b0, W_self1, W_neigh1, b1):
    raise NotImplementedError("write your pallas kernel here")


# speedup vs baseline: 5.3912x; 5.3912x over previous
import functools

import jax
import jax.numpy as jnp
from jax import lax
from jax.experimental import pallas as pl
from jax.experimental.pallas import tpu as pltpu
from jax.experimental.pallas import tpu_sc as plsc

NUM_NODES = 10000
NUM_EDGES = 320000
NC = 2
NS = 16
CH = 128
NCHUNK = 2560
EPAD = NCHUNK * CH
RP = 10240
RPT = RP // NS
HW = 64


def _zero_fill(ref, width):
    zv = jnp.zeros((16,), jnp.float32)

    def row(j, carry):
        for k in range(width // 16):
            ref[j, pl.ds(k * 16, 16)] = zv
        return carry

    lax.fori_loop(0, CH, row, 0)


_CPW_A = NCHUNK // NS

_mesh = plsc.VectorSubcoreMesh(core_axis_name="c", subcore_axis_name="s")


@functools.partial(
    pl.kernel,
    out_type=(
        jax.ShapeDtypeStruct((NC, RP, HW), jnp.float32),
        jax.ShapeDtypeStruct((NC, RP, 16), jnp.float32),
    ),
    mesh=_mesh,
    compiler_params=pltpu.CompilerParams(use_tc_tiling_on_sc=False),
    scratch_types=[
        pltpu.VMEM((_CPW_A, CH), jnp.int32),
        pltpu.VMEM((_CPW_A, CH), jnp.int32),
        pltpu.VMEM((CH, HW), jnp.float32),
        pltpu.VMEM((CH, HW), jnp.float32),
        pltpu.VMEM((CH, 16), jnp.float32),
        pltpu.VMEM((CH, 16), jnp.float32),
        pltpu.VMEM_SHARED((RP, HW), jnp.float32),
        pltpu.VMEM_SHARED((RP, 16), jnp.float32),
        pltpu.SemaphoreType.DMA,
    ],
)
def _sc_agg_l0(xcat_hbm, srcp2_hbm, dstp_hbm, out_hbm, deg_hbm,
               srcv, dstv, buf, zbuf, ones16, z16, acc, dacc, sem):
    c = lax.axis_index("c")
    s = lax.axis_index("s")

    _zero_fill(zbuf, HW)
    _zero_fill(z16, 16)
    ov = jnp.ones((16,), jnp.float32)

    def onesrow(j, carry):
        ones16[j, pl.ds(0, 16)] = ov
        return carry

    lax.fori_loop(0, CH, onesrow, 0)

    def zslab(k, carry):
        r = s * RPT + k * CH
        pltpu.sync_copy(zbuf, acc.at[pl.ds(r, CH)])
        pltpu.sync_copy(z16, dacc.at[pl.ds(r, CH)])
        return carry

    lax.fori_loop(0, RPT // CH, zslab, 0)
    plsc.subcore_barrier()

    base = s * _CPW_A
    pltpu.sync_copy(srcp2_hbm.at[c, pl.ds(base, _CPW_A)], srcv)
    pltpu.sync_copy(dstp_hbm.at[pl.ds(base, _CPW_A)], dstv)

    def chunk(j, carry):
        pltpu.async_copy(xcat_hbm.at[srcv.at[j]], buf, sem).wait()
        pltpu.sync_copy(buf, acc.at[dstv.at[j]], add=True)
        pltpu.sync_copy(ones16, dacc.at[dstv.at[j]], add=True)
        return carry

    lax.fori_loop(0, _CPW_A, chunk, 0)
    plsc.subcore_barrier()

    def oslab(k, carry):
        r = s * RPT + k * CH
        pltpu.sync_copy(acc.at[pl.ds(r, CH)], out_hbm.at[c, pl.ds(r, CH)])
        pltpu.sync_copy(dacc.at[pl.ds(r, CH)], deg_hbm.at[c, pl.ds(r, CH)])
        return carry

    lax.fori_loop(0, RPT // CH, oslab, 0)


_CPW_B = NCHUNK // (NC * NS)


@functools.partial(
    pl.kernel,
    out_type=jax.ShapeDtypeStruct((NC, RP, HW), jnp.float32),
    mesh=_mesh,
    compiler_params=pltpu.CompilerParams(use_tc_tiling_on_sc=False),
    scratch_types=[
        pltpu.VMEM((_CPW_B, CH), jnp.int32),
        pltpu.VMEM((_CPW_B, CH), jnp.int32),
        pltpu.VMEM((CH, HW), jnp.float32),
        pltpu.VMEM((CH, HW), jnp.float32),
        pltpu.VMEM_SHARED((RP, HW), jnp.float32),
        pltpu.SemaphoreType.DMA,
    ],
)
def _sc_agg_l1(z_hbm, srcp_hbm, dstp_hbm, out_hbm,
               srcv, dstv, buf, zbuf, acc, sem):
    c = lax.axis_index("c")
    s = lax.axis_index("s")
    wid = c * NS + s

    _zero_fill(zbuf, HW)

    def zslab(k, carry):
        r = s * RPT + k * CH
        pltpu.sync_copy(zbuf, acc.at[pl.ds(r, CH)])
        return carry

    lax.fori_loop(0, RPT // CH, zslab, 0)
    plsc.subcore_barrier()

    base = wid * _CPW_B
    pltpu.sync_copy(srcp_hbm.at[pl.ds(base, _CPW_B)], srcv)
    pltpu.sync_copy(dstp_hbm.at[pl.ds(base, _CPW_B)], dstv)

    def chunk(j, carry):
        pltpu.async_copy(z_hbm.at[srcv.at[j]], buf, sem).wait()
        pltpu.sync_copy(buf, acc.at[dstv.at[j]], add=True)
        return carry

    lax.fori_loop(0, _CPW_B, chunk, 0)
    plsc.subcore_barrier()

    def oslab(k, carry):
        r = s * RPT + k * CH
        pltpu.sync_copy(acc.at[pl.ds(r, CH)], out_hbm.at[c, pl.ds(r, CH)])
        return carry

    lax.fori_loop(0, RPT // CH, oslab, 0)


RB = 1000


def _tc1_body(x_ref, aggp_ref, degp_ref, ws0_ref, wn0_ref, b0_ref,
              ws1_ref, wn1_ref, b1_ref, z_ref, s_ref):
    agg = jnp.concatenate([aggp_ref[0], aggp_ref[1]], axis=-1)
    deg = degp_ref[0, :, 0:1]
    hn = agg / jnp.maximum(deg, 1.0)
    h1 = (jnp.dot(x_ref[...], ws0_ref[...], preferred_element_type=jnp.float32)
          + jnp.dot(hn, wn0_ref[...], preferred_element_type=jnp.float32)
          + b0_ref[...])
    h1 = jnp.maximum(h1, 0.0)
    z_ref[...] = jnp.dot(h1, wn1_ref[...], preferred_element_type=jnp.float32)
    s_ref[...] = (jnp.dot(h1, ws1_ref[...], preferred_element_type=jnp.float32)
                  + b1_ref[...])


def _tc2_body(s_ref, aggp_ref, degp_ref, out_ref):
    agg = aggp_ref[0] + aggp_ref[1]
    deg = degp_ref[0, :, 0:1]
    out_ref[...] = s_ref[...] + agg / jnp.maximum(deg, 1.0)


def kernel(x, edge_index, W_self0, W_neigh0, b0, W_self1, W_neigh1, b1):
    d_in = x.shape[1]
    d_hid = W_self0.shape[1]
    n_cls = W_self1.shape[1]

    src = edge_index[0]
    dst = edge_index[1]
    pad = EPAD - NUM_EDGES
    srcp = jnp.concatenate([src, jnp.zeros((pad,), jnp.int32)]).reshape(
        NCHUNK, CH)
    dstp = jnp.concatenate([dst, jnp.full((pad,), NUM_NODES, jnp.int32)]
                           ).reshape(NCHUNK, CH)
    srcp2 = jnp.stack([srcp, srcp + NUM_NODES])
    xcat = jnp.concatenate([x[:, :HW], x[:, HW:]], axis=0)

    agg0p, degp = _sc_agg_l0(xcat, srcp2, dstp)

    grid = (NUM_NODES // RB,)
    z, sflat = pl.pallas_call(
        _tc1_body,
        grid=grid,
        in_specs=[
            pl.BlockSpec((RB, d_in), lambda i: (i, 0)),
            pl.BlockSpec((NC, RB, HW), lambda i: (0, i, 0)),
            pl.BlockSpec((NC, RB, 16), lambda i: (0, i, 0)),
            pl.BlockSpec((d_in, d_hid), lambda i: (0, 0)),
            pl.BlockSpec((d_in, d_hid), lambda i: (0, 0)),
            pl.BlockSpec((1, d_hid), lambda i: (0, 0)),
            pl.BlockSpec((d_hid, n_cls), lambda i: (0, 0)),
            pl.BlockSpec((d_hid, n_cls), lambda i: (0, 0)),
            pl.BlockSpec((1, n_cls), lambda i: (0, 0)),
        ],
        out_specs=[
            pl.BlockSpec((RB, n_cls), lambda i: (i, 0)),
            pl.BlockSpec((RB, n_cls), lambda i: (i, 0)),
        ],
        out_shape=[
            jax.ShapeDtypeStruct((NUM_NODES, n_cls), jnp.float32),
            jax.ShapeDtypeStruct((NUM_NODES, n_cls), jnp.float32),
        ],
    )(x, agg0p, degp, W_self0, W_neigh0, b0[None, :], W_self1, W_neigh1,
      b1[None, :])

    agg1p = _sc_agg_l1(z, srcp, dstp)

    out = pl.pallas_call(
        _tc2_body,
        grid=grid,
        in_specs=[
            pl.BlockSpec((RB, n_cls), lambda i: (i, 0)),
            pl.BlockSpec((NC, RB, n_cls), lambda i: (0, i, 0)),
            pl.BlockSpec((NC, RB, 16), lambda i: (0, i, 0)),
        ],
        out_specs=pl.BlockSpec((RB, n_cls), lambda i: (i, 0)),
        out_shape=jax.ShapeDtypeStruct((NUM_NODES, n_cls), jnp.float32),
    )(sflat, agg1p, degp)

    return out
